# baseline (speedup 1.0000x reference)
import functools

import jax
import jax.numpy as jnp
from jax import lax
from jax.experimental import pallas as pl
from jax.experimental.pallas import tpu as pltpu
from jax.experimental.pallas import tpu_sc as plsc

T = 2048
E = 8
D = 768
F = 3072
KK = 2
A = T * KK
BT = 256
NB = A // BT + E - 1
PAD = NB * BT
NTILE = 16
SEG = A // NTILE
ZCH = PAD // NTILE

_MESH = plsc.VectorSubcoreMesh(core_axis_name="c", subcore_axis_name="s")


def _router_body(x_ref, wr_ref, br_ref, ti_ref, tv_ref):
    x = x_ref[...]
    logits = jnp.dot(x, wr_ref[...], preferred_element_type=jnp.float32)
    logits = logits + br_ref[...]
    lm = jnp.max(logits, axis=-1, keepdims=True)
    ex = jnp.exp(logits - lm)
    p = ex / jnp.sum(ex, axis=-1, keepdims=True)
    iota = lax.broadcasted_iota(jnp.int32, p.shape, 1)
    m1 = jnp.max(p, axis=-1, keepdims=True)
    i1 = jnp.min(jnp.where(p == m1, iota, E), axis=-1, keepdims=True)
    pneg = jnp.where(iota == i1, -jnp.inf, p)
    m2 = jnp.max(pneg, axis=-1, keepdims=True)
    i2 = jnp.min(jnp.where(pneg == m2, iota, E), axis=-1, keepdims=True)
    ti_ref[...] = jnp.concatenate([i1, i2], axis=1)
    tv_ref[...] = jnp.concatenate([m1, m2], axis=1)


def _router(x, Wr, br2):
    nt = T // BT
    return pl.pallas_call(
        _router_body,
        grid=(nt,),
        in_specs=[
            pl.BlockSpec((BT, D), lambda t: (t, 0)),
            pl.BlockSpec((D, E), lambda t: (0, 0)),
            pl.BlockSpec((1, E), lambda t: (0, 0)),
        ],
        out_specs=[
            pl.BlockSpec((BT, KK), lambda t: (t, 0)),
            pl.BlockSpec((BT, KK), lambda t: (t, 0)),
        ],
        out_shape=[
            jax.ShapeDtypeStruct((T, KK), jnp.int32),
            jax.ShapeDtypeStruct((T, KK), jnp.float32),
        ],
    )(x, Wr, br2)


@functools.partial(
    pl.kernel,
    out_type=(
        jax.ShapeDtypeStruct((PAD,), jnp.int32),
        jax.ShapeDtypeStruct((PAD,), jnp.float32),
        jax.ShapeDtypeStruct((T,), jnp.int32),
        jax.ShapeDtypeStruct((T,), jnp.int32),
        jax.ShapeDtypeStruct((32,), jnp.int32),
        jax.ShapeDtypeStruct((NTILE, 16), jnp.int32),
    ),
    mesh=_MESH,
    compiler_params=pltpu.CompilerParams(needs_layout_passes=False),
    scratch_types=[
        pltpu.VMEM((SEG,), jnp.int32),
        pltpu.VMEM((2, SEG // 2), jnp.float32),
        pltpu.VMEM((16,), jnp.int32),
        pltpu.VMEM((NTILE, 16), jnp.int32),
        pltpu.VMEM((SEG,), jnp.int32),
        pltpu.VMEM((SEG,), jnp.int32),
        pltpu.VMEM((2, SEG // 2), jnp.int32),
        pltpu.VMEM((2, SEG // 2), jnp.int32),
        pltpu.VMEM((SEG // 2,), jnp.int32),
        pltpu.VMEM((SEG // 2,), jnp.int32),
        pltpu.VMEM((ZCH,), jnp.int32),
        pltpu.VMEM((ZCH,), jnp.float32),
        pltpu.VMEM((32,), jnp.int32),
    ],
)
def _dispatch(ei_hbm, gv_hbm, idxs_hbm, gs_hbm, pos0_hbm, pos1_hbm, sinfo_hbm,
              cnt_hbm, eiv, gvv, cnt_loc, cnts, tokl, posl, toks2, poss2,
              p0l, p1l, zbi, zbf, sinfo_loc):
    cid = lax.axis_index("c")
    sid = lax.axis_index("s")

    @pl.when(cid == 0)
    def _():
        iota = lax.iota(jnp.int32, 16)
        for k in range(ZCH // 16):
            zbi[pl.ds(k * 16, 16)] = jnp.zeros((16,), jnp.int32)
            zbf[pl.ds(k * 16, 16)] = jnp.zeros((16,), jnp.float32)
        pltpu.sync_copy(zbi, idxs_hbm.at[pl.ds(sid * ZCH, ZCH)])
        pltpu.sync_copy(zbf, gs_hbm.at[pl.ds(sid * ZCH, ZCH)])
        pltpu.sync_copy(ei_hbm.at[pl.ds(sid * SEG, SEG)], eiv)
        for j in range(2):
            pltpu.sync_copy(gv_hbm.at[pl.ds(sid * SEG + j * 128, 128)],
                            gvv.at[j])

        def _count_chunk(c, cntv):
            v = eiv[pl.ds(c * 16, 16)]
            for e in range(E):
                pc = plsc.all_reduce_population_count(v == e)
                cntv = cntv + jnp.where(iota == e, pc, 0)
            return cntv
        cnt_loc[...] = lax.fori_loop(0, SEG // 16, _count_chunk,
                                     jnp.zeros((16,), jnp.int32))
        pltpu.sync_copy(cnt_loc, cnt_hbm.at[sid])
        plsc.subcore_barrier()
        pltpu.sync_copy(cnt_hbm, cnts)

        tot = jnp.zeros((16,), jnp.int32)
        pre = jnp.zeros((16,), jnp.int32)
        for r in range(NTILE):
            row = cnts[r, :]
            tot = tot + row
            pre = pre + jnp.where(jnp.full((16,), r, jnp.int32) < sid, row, 0)
        n = [tot[e] for e in range(E)]
        pr = [pre[e] for e in range(E)]
        sp = [jnp.int32(0)]
        for e in range(E):
            nblk = (n[e] + (BT - 1)) // BT
            sp.append(sp[e] + nblk * BT)
        offs = [sp[e] + pr[e] for e in range(E)]
        nbu = sp[E] // BT

        def _pos_chunk(c, offs):
            offs = list(offs)
            v = eiv[pl.ds(c * 16, 16)]
            pos = jnp.zeros((16,), jnp.int32)
            for e in range(E):
                m = v == e
                pc = plsc.cumsum(jnp.where(m, 1, 0).astype(jnp.int32))
                pos = jnp.where(m, offs[e] + pc - 1, pos)
                cnt = plsc.all_reduce_population_count(m)
                offs[e] = offs[e] + cnt[0]
            pos = jnp.clip(pos, 0, PAD - 1)
            gi = sid * SEG + c * 16 + iota
            tokl[pl.ds(c * 16, 16)] = gi >> 1
            posl[pl.ds(c * 16, 16)] = pos
            lt = (c * 16 + iota) >> 1
            even = (gi & 1) == 0
            plsc.store_scatter(p0l, [lt], pos, mask=even)
            plsc.store_scatter(p1l, [lt], pos, mask=jnp.logical_not(even))
            return tuple(offs)
        lax.fori_loop(0, SEG // 16, _pos_chunk, tuple(offs))

        for j in range(2):
            for k in range(8):
                s = pl.ds(j * 128 + k * 16, 16)
                toks2[j, pl.ds(k * 16, 16)] = tokl[s]
                poss2[j, pl.ds(k * 16, 16)] = posl[s]
        for j in range(2):
            pltpu.sync_copy(toks2.at[j], idxs_hbm.at[poss2.at[j]])
            pltpu.sync_copy(gvv.at[j], gs_hbm.at[poss2.at[j]])
        pltpu.sync_copy(p0l, pos0_hbm.at[pl.ds(sid * (SEG // 2), SEG // 2)])
        pltpu.sync_copy(p1l, pos1_hbm.at[pl.ds(sid * (SEG // 2), SEG // 2)])

        @pl.when(sid == 0)
        def _():
            for c2 in range(2):
                bstart = (c2 * 16 + iota) * BT
                acc = jnp.zeros((16,), jnp.int32)
                for e in range(E):
                    acc = acc + (bstart >= sp[e + 1]).astype(jnp.int32)
                be = jnp.minimum(acc, E - 1)
                if c2 == 1:
                    be = jnp.where(iota == 15, nbu, be)
                sinfo_loc[pl.ds(c2 * 16, 16)] = be
            pltpu.sync_copy(sinfo_loc, sinfo_hbm)


@functools.partial(
    pl.kernel,
    out_type=jax.ShapeDtypeStruct((PAD, D), jnp.float32),
    mesh=_MESH,
    compiler_params=pltpu.CompilerParams(needs_layout_passes=False),
    scratch_types=[
        pltpu.VMEM((96,), jnp.int32),
        pltpu.VMEM((88,), jnp.int32),
        pltpu.VMEM((96, D), jnp.float32),
        pltpu.SemaphoreType.DMA,
    ],
)
def _gather_rows(x_hbm, idxs_hbm, xs_hbm, idxa, idxb, rows, sem):
    wid = lax.axis_index("s") * 2 + lax.axis_index("c")
    r0 = wid * (PAD // 32)
    pltpu.sync_copy(idxs_hbm.at[pl.ds(r0, 96)], idxa)
    pltpu.sync_copy(idxs_hbm.at[pl.ds(r0 + 96, 88)], idxb)
    pltpu.async_copy(x_hbm.at[idxa], rows, sem).wait()
    pltpu.sync_copy(rows, xs_hbm.at[pl.ds(r0, 96)])
    pltpu.async_copy(x_hbm.at[idxb], rows.at[pl.ds(0, 88)], sem).wait()
    pltpu.sync_copy(rows.at[pl.ds(0, 88)], xs_hbm.at[pl.ds(r0 + 96, 88)])


def _gmm_body(sref, xs_ref, w1_ref, b1_ref, w2_ref, b2_ref, gs_ref, ys_ref):
    b = pl.program_id(0)

    @pl.when(b < sref[31])
    def _():
        h = jnp.dot(xs_ref[...], w1_ref[0], preferred_element_type=jnp.float32)
        h = jnp.maximum(h + b1_ref[0], 0.0)
        y = jnp.dot(h, w2_ref[0], preferred_element_type=jnp.float32)
        ys_ref[...] = gs_ref[...] * (y + b2_ref[0])


def _gmm(sinfo, xs, W1, b1r, W2, b2r, gs2):
    grid_spec = pltpu.PrefetchScalarGridSpec(
        num_scalar_prefetch=1,
        grid=(NB,),
        in_specs=[
            pl.BlockSpec((BT, D), lambda b, s: (b, 0)),
            pl.BlockSpec((1, D, F), lambda b, s: (s[b], 0, 0)),
            pl.BlockSpec((1, 1, F), lambda b, s: (s[b], 0, 0)),
            pl.BlockSpec((1, F, D), lambda b, s: (s[b], 0, 0)),
            pl.BlockSpec((1, 1, D), lambda b, s: (s[b], 0, 0)),
            pl.BlockSpec((BT, 1), lambda b, s: (b, 0)),
        ],
        out_specs=pl.BlockSpec((BT, D), lambda b, s: (b, 0)),
    )
    return pl.pallas_call(
        _gmm_body,
        grid_spec=grid_spec,
        out_shape=jax.ShapeDtypeStruct((PAD, D), jnp.float32),
    )(sinfo, xs, W1, b1r, W2, b2r, gs2)


@functools.partial(
    pl.kernel,
    out_type=jax.ShapeDtypeStruct((T, D), jnp.float32),
    mesh=_MESH,
    compiler_params=pltpu.CompilerParams(needs_layout_passes=False),
    scratch_types=[
        pltpu.VMEM((T // 32,), jnp.int32),
        pltpu.VMEM((T // 32,), jnp.int32),
        pltpu.VMEM((T // 32, D), jnp.float32),
        pltpu.VMEM((T // 32, D), jnp.float32),
        pltpu.SemaphoreType.DMA,
        pltpu.SemaphoreType.DMA,
    ],
)
def _combine(ys_hbm, pos0_hbm, pos1_hbm, out_hbm,
             idx0, idx1, buf0, buf1, sem0, sem1):
    wid = lax.axis_index("s") * 2 + lax.axis_index("c")
    nt = T // 32
    t0 = wid * nt
    pltpu.sync_copy(pos0_hbm.at[pl.ds(t0, nt)], idx0)
    pltpu.sync_copy(pos1_hbm.at[pl.ds(t0, nt)], idx1)
    c0 = pltpu.async_copy(ys_hbm.at[idx0], buf0, sem0)
    c1 = pltpu.async_copy(ys_hbm.at[idx1], buf1, sem1)
    c0.wait()
    c1.wait()

    def row(r, _):
        def chunk(k, __):
            s = pl.ds(k * 16, 16)
            buf0[r, s] = buf0[r, s] + buf1[r, s]
            return __
        lax.fori_loop(0, D // 16, chunk, 0)
        return _
    lax.fori_loop(0, nt, row, 0)
    pltpu.sync_copy(buf0, out_hbm.at[pl.ds(t0, nt)])


def kernel(x, Wr, br, W1, b1, W2, b2):
    topi, topv = _router(x, Wr, br.reshape(1, E))
    ei = topi.reshape(-1)
    gv = topv.reshape(-1)
    idxs, gs, pos0, pos1, sinfo, _ = _dispatch(ei, gv)
    xs = _gather_rows(x, idxs)
    ys = _gmm(sinfo, xs, W1, b1.reshape(E, 1, F), W2, b2.reshape(E, 1, D),
              gs.reshape(PAD, 1))
    return _combine(ys, pos0, pos1)


# speedup vs baseline: 1.0129x; 1.0129x over previous
import functools

import jax
import jax.numpy as jnp
from jax import lax
from jax.experimental import pallas as pl
from jax.experimental.pallas import tpu as pltpu
from jax.experimental.pallas import tpu_sc as plsc

T = 2048
E = 8
D = 768
F = 3072
KK = 2
A = T * KK
BT = 256
NB = A // BT + E - 1
PAD = NB * BT
PAD2 = PAD + 64
NTILE = 16
SEG = A // NTILE
RPW = PAD // 32

_MESH = plsc.VectorSubcoreMesh(core_axis_name="c", subcore_axis_name="s")


def _router_body(x_ref, wr_ref, br_ref, ti_ref, tv_ref):
    x = x_ref[...]
    logits = jnp.dot(x, wr_ref[...], preferred_element_type=jnp.float32)
    logits = logits + br_ref[...]
    lm = jnp.max(logits, axis=-1, keepdims=True)
    ex = jnp.exp(logits - lm)
    p = ex / jnp.sum(ex, axis=-1, keepdims=True)
    iota = lax.broadcasted_iota(jnp.int32, p.shape, 1)
    m1 = jnp.max(p, axis=-1, keepdims=True)
    i1 = jnp.min(jnp.where(p == m1, iota, E), axis=-1, keepdims=True)
    pneg = jnp.where(iota == i1, -jnp.inf, p)
    m2 = jnp.max(pneg, axis=-1, keepdims=True)
    i2 = jnp.min(jnp.where(pneg == m2, iota, E), axis=-1, keepdims=True)
    ti_ref[...] = jnp.concatenate([i1, i2], axis=1)
    tv_ref[...] = jnp.concatenate([m1, m2], axis=1)


def _router(x, Wr, br2):
    nt = T // BT
    return pl.pallas_call(
        _router_body,
        grid=(nt,),
        in_specs=[
            pl.BlockSpec((BT, D), lambda t: (t, 0)),
            pl.BlockSpec((D, E), lambda t: (0, 0)),
            pl.BlockSpec((1, E), lambda t: (0, 0)),
        ],
        out_specs=[
            pl.BlockSpec((BT, KK), lambda t: (t, 0)),
            pl.BlockSpec((BT, KK), lambda t: (t, 0)),
        ],
        out_shape=[
            jax.ShapeDtypeStruct((T, KK), jnp.int32),
            jax.ShapeDtypeStruct((T, KK), jnp.float32),
        ],
    )(x, Wr, br2)


@functools.partial(
    pl.kernel,
    out_type=(
        jax.ShapeDtypeStruct((PAD2,), jnp.int32),
        jax.ShapeDtypeStruct((PAD,), jnp.float32),
        jax.ShapeDtypeStruct((T,), jnp.int32),
        jax.ShapeDtypeStruct((T,), jnp.int32),
        jax.ShapeDtypeStruct((32,), jnp.int32),
        jax.ShapeDtypeStruct((NTILE, 16), jnp.int32),
    ),
    mesh=_MESH,
    compiler_params=pltpu.CompilerParams(needs_layout_passes=False),
    scratch_types=[
        pltpu.VMEM((SEG,), jnp.int32),
        pltpu.VMEM((2, SEG // 2), jnp.float32),
        pltpu.VMEM((16,), jnp.int32),
        pltpu.VMEM((NTILE, 16), jnp.int32),
        pltpu.VMEM((SEG,), jnp.int32),
        pltpu.VMEM((SEG,), jnp.int32),
        pltpu.VMEM((2, SEG // 2), jnp.int32),
        pltpu.VMEM((2, SEG // 2), jnp.int32),
        pltpu.VMEM((SEG // 2,), jnp.int32),
        pltpu.VMEM((SEG // 2,), jnp.int32),
        pltpu.VMEM((32,), jnp.int32),
    ],
)
def _dispatch(ei_hbm, gv_hbm, idxs_hbm, gs_hbm, pos0_hbm, pos1_hbm, sinfo_hbm,
              cnt_hbm, eiv, gvv, cnt_loc, cnts, tokl, posl, toks2, poss2,
              p0l, p1l, sinfo_loc):
    cid = lax.axis_index("c")
    sid = lax.axis_index("s")

    @pl.when(cid == 0)
    def _():
        iota = lax.iota(jnp.int32, 16)
        pltpu.sync_copy(ei_hbm.at[pl.ds(sid * SEG, SEG)], eiv)
        for j in range(2):
            pltpu.sync_copy(gv_hbm.at[pl.ds(sid * SEG + j * 128, 128)],
                            gvv.at[j])

        def _count_chunk(c, cntv):
            v = eiv[pl.ds(c * 16, 16)]
            for e in range(E):
                pc = plsc.all_reduce_population_count(v == e)
                cntv = cntv + jnp.where(iota == e, pc, 0)
            return cntv
        cnt_loc[...] = lax.fori_loop(0, SEG // 16, _count_chunk,
                                     jnp.zeros((16,), jnp.int32))
        pltpu.sync_copy(cnt_loc, cnt_hbm.at[sid])
        plsc.subcore_barrier()
        pltpu.sync_copy(cnt_hbm, cnts)

        tot = jnp.zeros((16,), jnp.int32)
        pre = jnp.zeros((16,), jnp.int32)
        for r in range(NTILE):
            row = cnts[r, :]
            tot = tot + row
            pre = pre + jnp.where(jnp.full((16,), r, jnp.int32) < sid, row, 0)
        n = [tot[e] for e in range(E)]
        pr = [pre[e] for e in range(E)]
        sp = [jnp.int32(0)]
        for e in range(E):
            nblk = (n[e] + (BT - 1)) // BT
            sp.append(sp[e] + nblk * BT)
        offs = [sp[e] + pr[e] for e in range(E)]
        nbu = sp[E] // BT

        def _pos_chunk(c, offs):
            offs = list(offs)
            v = eiv[pl.ds(c * 16, 16)]
            pos = jnp.zeros((16,), jnp.int32)
            for e in range(E):
                m = v == e
                pc = plsc.cumsum(jnp.where(m, 1, 0).astype(jnp.int32))
                pos = jnp.where(m, offs[e] + pc - 1, pos)
                cnt = plsc.all_reduce_population_count(m)
                offs[e] = offs[e] + cnt[0]
            pos = jnp.clip(pos, 0, PAD - 1)
            gi = sid * SEG + c * 16 + iota
            tokl[pl.ds(c * 16, 16)] = gi >> 1
            posl[pl.ds(c * 16, 16)] = pos
            lt = (c * 16 + iota) >> 1
            even = (gi & 1) == 0
            plsc.store_scatter(p0l, [lt], pos, mask=even)
            plsc.store_scatter(p1l, [lt], pos, mask=jnp.logical_not(even))
            return tuple(offs)
        lax.fori_loop(0, SEG // 16, _pos_chunk, tuple(offs))

        for j in range(2):
            for k in range(8):
                s = pl.ds(j * 128 + k * 16, 16)
                toks2[j, pl.ds(k * 16, 16)] = tokl[s]
                poss2[j, pl.ds(k * 16, 16)] = posl[s]
        for j in range(2):
            pltpu.sync_copy(toks2.at[j], idxs_hbm.at[poss2.at[j]])
            pltpu.sync_copy(gvv.at[j], gs_hbm.at[poss2.at[j]])
        pltpu.sync_copy(p0l, pos0_hbm.at[pl.ds(sid * (SEG // 2), SEG // 2)])
        pltpu.sync_copy(p1l, pos1_hbm.at[pl.ds(sid * (SEG // 2), SEG // 2)])

        @pl.when(sid == 0)
        def _():
            for c2 in range(2):
                bstart = (c2 * 16 + iota) * BT
                acc = jnp.zeros((16,), jnp.int32)
                for e in range(E):
                    acc = acc + (bstart >= sp[e + 1]).astype(jnp.int32)
                be = jnp.minimum(acc, E - 1)
                if c2 == 1:
                    be = jnp.where(iota == 15, nbu, be)
                sinfo_loc[pl.ds(c2 * 16, 16)] = be
            pltpu.sync_copy(sinfo_loc, sinfo_hbm)


@functools.partial(
    pl.kernel,
    out_type=jax.ShapeDtypeStruct((PAD2, D), jnp.float32),
    mesh=_MESH,
    compiler_params=pltpu.CompilerParams(needs_layout_passes=False),
    scratch_types=[
        pltpu.VMEM((3, 64), jnp.int32),
        pltpu.VMEM((64, D), jnp.float32),
        pltpu.VMEM((64, D), jnp.float32),
        pltpu.SemaphoreType.DMA,
        pltpu.SemaphoreType.DMA,
    ],
)
def _gather_rows(x_hbm, idxs_hbm, xs_hbm, idxm, buf0, buf1, sem0, sem1):
    wid = lax.axis_index("s") * 2 + lax.axis_index("c")
    r0 = wid * RPW
    for j in range(3):
        pltpu.sync_copy(idxs_hbm.at[pl.ds(r0 + 64 * j, 64)], idxm.at[j])
        for k in range(4):
            s = pl.ds(k * 16, 16)
            idxm[j, s] = jnp.clip(idxm[j, s], 0, T - 1)
    g0 = pltpu.async_copy(x_hbm.at[idxm.at[0]], buf0, sem0)
    g1 = pltpu.async_copy(x_hbm.at[idxm.at[1]], buf1, sem1)
    g0.wait()
    pltpu.sync_copy(buf0, xs_hbm.at[pl.ds(r0, 64)])
    g2 = pltpu.async_copy(x_hbm.at[idxm.at[2]], buf0, sem0)
    g1.wait()
    pltpu.sync_copy(buf1, xs_hbm.at[pl.ds(r0 + 64, 64)])
    g2.wait()
    pltpu.sync_copy(buf0, xs_hbm.at[pl.ds(r0 + 128, 64)])


def _gmm_body(sref, xs_ref, w1_ref, b1_ref, w2_ref, b2_ref, gs_ref, ys_ref):
    b = pl.program_id(0)

    @pl.when(b < sref[31])
    def _():
        h = jnp.dot(xs_ref[...], w1_ref[0], preferred_element_type=jnp.float32)
        h = jnp.maximum(h + b1_ref[0], 0.0)
        y = jnp.dot(h, w2_ref[0], preferred_element_type=jnp.float32)
        ys_ref[...] = gs_ref[...] * (y + b2_ref[0])


def _gmm(sinfo, xs, W1, b1r, W2, b2r, gs2):
    grid_spec = pltpu.PrefetchScalarGridSpec(
        num_scalar_prefetch=1,
        grid=(NB,),
        in_specs=[
            pl.BlockSpec((BT, D), lambda b, s: (b, 0)),
            pl.BlockSpec((1, D, F), lambda b, s: (s[b], 0, 0)),
            pl.BlockSpec((1, 1, F), lambda b, s: (s[b], 0, 0)),
            pl.BlockSpec((1, F, D), lambda b, s: (s[b], 0, 0)),
            pl.BlockSpec((1, 1, D), lambda b, s: (s[b], 0, 0)),
            pl.BlockSpec((BT, 1), lambda b, s: (b, 0)),
        ],
        out_specs=pl.BlockSpec((BT, D), lambda b, s: (b, 0)),
    )
    return pl.pallas_call(
        _gmm_body,
        grid_spec=grid_spec,
        out_shape=jax.ShapeDtypeStruct((PAD, D), jnp.float32),
    )(sinfo, xs, W1, b1r, W2, b2r, gs2)


@functools.partial(
    pl.kernel,
    out_type=jax.ShapeDtypeStruct((T, D), jnp.float32),
    mesh=_MESH,
    compiler_params=pltpu.CompilerParams(needs_layout_passes=False),
    scratch_types=[
        pltpu.VMEM((T // 32,), jnp.int32),
        pltpu.VMEM((T // 32,), jnp.int32),
        pltpu.VMEM((T // 32, D), jnp.float32),
        pltpu.VMEM((T // 32, D), jnp.float32),
        pltpu.SemaphoreType.DMA,
        pltpu.SemaphoreType.DMA,
    ],
)
def _combine(ys_hbm, pos0_hbm, pos1_hbm, out_hbm,
             idx0, idx1, buf0, buf1, sem0, sem1):
    wid = lax.axis_index("s") * 2 + lax.axis_index("c")
    nt = T // 32
    t0 = wid * nt
    pltpu.sync_copy(pos0_hbm.at[pl.ds(t0, nt)], idx0)
    pltpu.sync_copy(pos1_hbm.at[pl.ds(t0, nt)], idx1)
    c0 = pltpu.async_copy(ys_hbm.at[idx0], buf0, sem0)
    c1 = pltpu.async_copy(ys_hbm.at[idx1], buf1, sem1)
    c0.wait()
    c1.wait()

    def row(r, _):
        for k in range(D // 16):
            s = pl.ds(k * 16, 16)
            buf0[r, s] = buf0[r, s] + buf1[r, s]
        return _
    lax.fori_loop(0, nt, row, 0)
    pltpu.sync_copy(buf0, out_hbm.at[pl.ds(t0, nt)])


def kernel(x, Wr, br, W1, b1, W2, b2):
    topi, topv = _router(x, Wr, br.reshape(1, E))
    ei = topi.reshape(-1)
    gv = topv.reshape(-1)
    idxs, gs, pos0, pos1, sinfo, _ = _dispatch(ei, gv)
    xs = _gather_rows(x, idxs)
    ys = _gmm(sinfo, xs, W1, b1.reshape(E, 1, F), W2, b2.reshape(E, 1, D),
              gs.reshape(PAD, 1))
    return _combine(ys, pos0, pos1)


# speedup vs baseline: 1.3133x; 1.2967x over previous
import functools

import jax
import jax.numpy as jnp
from jax import lax
from jax.experimental import pallas as pl
from jax.experimental.pallas import tpu as pltpu
from jax.experimental.pallas import tpu_sc as plsc

T = 2048
E = 8
D = 768
F = 3072
KK = 2
A = T * KK
BT = 256
NB = A // BT + E - 1
PAD = NB * BT
PAD2 = PAD + 64
NTILE = 16
SEG = A // NTILE
RPW = PAD // 32

_MESH = plsc.VectorSubcoreMesh(core_axis_name="c", subcore_axis_name="s")


def _router_body(x_ref, wr_ref, br_ref, ti_ref, tv_ref):
    x = x_ref[...]
    logits = jnp.dot(x, wr_ref[...], preferred_element_type=jnp.float32)
    logits = logits + br_ref[...]
    lm = jnp.max(logits, axis=-1, keepdims=True)
    ex = jnp.exp(logits - lm)
    p = ex / jnp.sum(ex, axis=-1, keepdims=True)
    iota = lax.broadcasted_iota(jnp.int32, p.shape, 1)
    m1 = jnp.max(p, axis=-1, keepdims=True)
    i1 = jnp.min(jnp.where(p == m1, iota, E), axis=-1, keepdims=True)
    pneg = jnp.where(iota == i1, -jnp.inf, p)
    m2 = jnp.max(pneg, axis=-1, keepdims=True)
    i2 = jnp.min(jnp.where(pneg == m2, iota, E), axis=-1, keepdims=True)
    ti_ref[...] = jnp.concatenate([i1, i2], axis=1)
    tv_ref[...] = jnp.concatenate([m1, m2], axis=1)


def _router(x, Wr, br2):
    nt = T // BT
    return pl.pallas_call(
        _router_body,
        grid=(nt,),
        in_specs=[
            pl.BlockSpec((BT, D), lambda t: (t, 0)),
            pl.BlockSpec((D, E), lambda t: (0, 0)),
            pl.BlockSpec((1, E), lambda t: (0, 0)),
        ],
        out_specs=[
            pl.BlockSpec((BT, KK), lambda t: (t, 0)),
            pl.BlockSpec((BT, KK), lambda t: (t, 0)),
        ],
        out_shape=[
            jax.ShapeDtypeStruct((T, KK), jnp.int32),
            jax.ShapeDtypeStruct((T, KK), jnp.float32),
        ],
    )(x, Wr, br2)


@functools.partial(
    pl.kernel,
    out_type=(
        jax.ShapeDtypeStruct((PAD2,), jnp.int32),
        jax.ShapeDtypeStruct((PAD,), jnp.float32),
        jax.ShapeDtypeStruct((T,), jnp.int32),
        jax.ShapeDtypeStruct((T,), jnp.int32),
        jax.ShapeDtypeStruct((32,), jnp.int32),
        jax.ShapeDtypeStruct((NTILE, 16), jnp.int32),
    ),
    mesh=_MESH,
    compiler_params=pltpu.CompilerParams(needs_layout_passes=False),
    scratch_types=[
        pltpu.VMEM((SEG,), jnp.int32),
        pltpu.VMEM((2, SEG // 2), jnp.float32),
        pltpu.VMEM((16,), jnp.int32),
        pltpu.VMEM((NTILE, 16), jnp.int32),
        pltpu.VMEM((SEG,), jnp.int32),
        pltpu.VMEM((SEG,), jnp.int32),
        pltpu.VMEM((2, SEG // 2), jnp.int32),
        pltpu.VMEM((2, SEG // 2), jnp.int32),
        pltpu.VMEM((SEG // 2,), jnp.int32),
        pltpu.VMEM((SEG // 2,), jnp.int32),
        pltpu.VMEM((32,), jnp.int32),
    ],
)
def _dispatch(ei_hbm, gv_hbm, idxs_hbm, gs_hbm, pos0_hbm, pos1_hbm, sinfo_hbm,
              cnt_hbm, eiv, gvv, cnt_loc, cnts, tokl, posl, toks2, poss2,
              p0l, p1l, sinfo_loc):
    cid = lax.axis_index("c")
    sid = lax.axis_index("s")

    @pl.when(cid == 0)
    def _():
        iota = lax.iota(jnp.int32, 16)
        pltpu.sync_copy(ei_hbm.at[pl.ds(sid * SEG, SEG)], eiv)
        for j in range(2):
            pltpu.sync_copy(gv_hbm.at[pl.ds(sid * SEG + j * 128, 128)],
                            gvv.at[j])

        def _count_chunk(c, cntv):
            v = eiv[pl.ds(c * 16, 16)]
            for e in range(E):
                pc = plsc.all_reduce_population_count(v == e)
                cntv = cntv + jnp.where(iota == e, pc, 0)
            return cntv
        cnt_loc[...] = lax.fori_loop(0, SEG // 16, _count_chunk,
                                     jnp.zeros((16,), jnp.int32))
        pltpu.sync_copy(cnt_loc, cnt_hbm.at[sid])
        plsc.subcore_barrier()
        pltpu.sync_copy(cnt_hbm, cnts)

        tot = jnp.zeros((16,), jnp.int32)
        pre = jnp.zeros((16,), jnp.int32)
        for r in range(NTILE):
            row = cnts[r, :]
            tot = tot + row
            pre = pre + jnp.where(jnp.full((16,), r, jnp.int32) < sid, row, 0)
        n = [tot[e] for e in range(E)]
        pr = [pre[e] for e in range(E)]
        sp = [jnp.int32(0)]
        for e in range(E):
            nblk = (n[e] + (BT - 1)) // BT
            sp.append(sp[e] + nblk * BT)
        offs = [sp[e] + pr[e] for e in range(E)]
        nbu = sp[E] // BT

        def _pos_chunk(c, offs):
            offs = list(offs)
            v = eiv[pl.ds(c * 16, 16)]
            pos = jnp.zeros((16,), jnp.int32)
            for e in range(E):
                m = v == e
                pc = plsc.cumsum(jnp.where(m, 1, 0).astype(jnp.int32))
                pos = jnp.where(m, offs[e] + pc - 1, pos)
                cnt = plsc.all_reduce_population_count(m)
                offs[e] = offs[e] + cnt[0]
            pos = jnp.clip(pos, 0, PAD - 1)
            gi = sid * SEG + c * 16 + iota
            tokl[pl.ds(c * 16, 16)] = gi >> 1
            posl[pl.ds(c * 16, 16)] = pos
            lt = (c * 16 + iota) >> 1
            even = (gi & 1) == 0
            plsc.store_scatter(p0l, [lt], pos, mask=even)
            plsc.store_scatter(p1l, [lt], pos, mask=jnp.logical_not(even))
            return tuple(offs)
        lax.fori_loop(0, SEG // 16, _pos_chunk, tuple(offs))

        for j in range(2):
            for k in range(8):
                s = pl.ds(j * 128 + k * 16, 16)
                toks2[j, pl.ds(k * 16, 16)] = tokl[s]
                poss2[j, pl.ds(k * 16, 16)] = posl[s]
        for j in range(2):
            pltpu.sync_copy(toks2.at[j], idxs_hbm.at[poss2.at[j]])
            pltpu.sync_copy(gvv.at[j], gs_hbm.at[poss2.at[j]])
        pltpu.sync_copy(p0l, pos0_hbm.at[pl.ds(sid * (SEG // 2), SEG // 2)])
        pltpu.sync_copy(p1l, pos1_hbm.at[pl.ds(sid * (SEG // 2), SEG // 2)])

        @pl.when(sid == 0)
        def _():
            for c2 in range(2):
                bstart = (c2 * 16 + iota) * BT
                acc = jnp.zeros((16,), jnp.int32)
                for e in range(E):
                    acc = acc + (bstart >= sp[e + 1]).astype(jnp.int32)
                be = jnp.minimum(acc, E - 1)
                if c2 == 1:
                    be = jnp.where(iota == 15, nbu, be)
                sinfo_loc[pl.ds(c2 * 16, 16)] = be
            pltpu.sync_copy(sinfo_loc, sinfo_hbm)


def _gmm_body(sref, idx_ref, xh_ref, w1_ref, b1_ref, w2_ref, b2_ref, gs_ref,
              ys_ref):
    b = pl.program_id(0)

    @pl.when(b < sref[31])
    def _():
        idx = idx_ref[0]
        iota = lax.broadcasted_iota(jnp.int32, (BT, T), 1)
        oh = (idx == iota).astype(jnp.bfloat16)
        xsel = jnp.dot(oh, xh_ref[...],
                       preferred_element_type=jnp.float32)
        h = jnp.dot(xsel, w1_ref[0], preferred_element_type=jnp.float32)
        h = jnp.maximum(h + b1_ref[0], 0.0)
        y = jnp.dot(h, w2_ref[0], preferred_element_type=jnp.float32)
        ys_ref[...] = gs_ref[...] * (y + b2_ref[0])


def _gmm(sinfo, idxs3, xh, W1, b1r, W2, b2r, gs2):
    grid_spec = pltpu.PrefetchScalarGridSpec(
        num_scalar_prefetch=1,
        grid=(NB,),
        in_specs=[
            pl.BlockSpec((1, BT, 1), lambda b, s: (b, 0, 0)),
            pl.BlockSpec((T, D), lambda b, s: (0, 0)),
            pl.BlockSpec((1, D, F), lambda b, s: (s[b], 0, 0)),
            pl.BlockSpec((1, 1, F), lambda b, s: (s[b], 0, 0)),
            pl.BlockSpec((1, F, D), lambda b, s: (s[b], 0, 0)),
            pl.BlockSpec((1, 1, D), lambda b, s: (s[b], 0, 0)),
            pl.BlockSpec((BT, 1), lambda b, s: (b, 0)),
        ],
        out_specs=pl.BlockSpec((BT, D), lambda b, s: (b, 0)),
    )
    return pl.pallas_call(
        _gmm_body,
        grid_spec=grid_spec,
        out_shape=jax.ShapeDtypeStruct((PAD, D), jnp.float32),
    )(sinfo, idxs3, xh, W1, b1r, W2, b2r, gs2)


@functools.partial(
    pl.kernel,
    out_type=jax.ShapeDtypeStruct((T, D), jnp.float32),
    mesh=_MESH,
    compiler_params=pltpu.CompilerParams(needs_layout_passes=False),
    scratch_types=[
        pltpu.VMEM((T // 32,), jnp.int32),
        pltpu.VMEM((T // 32,), jnp.int32),
        pltpu.VMEM((T // 32, D), jnp.float32),
        pltpu.VMEM((T // 32, D), jnp.float32),
        pltpu.SemaphoreType.DMA,
        pltpu.SemaphoreType.DMA,
    ],
)
def _combine(ys_hbm, pos0_hbm, pos1_hbm, out_hbm,
             idx0, idx1, buf0, buf1, sem0, sem1):
    wid = lax.axis_index("s") * 2 + lax.axis_index("c")
    nt = T // 32
    t0 = wid * nt
    pltpu.sync_copy(pos0_hbm.at[pl.ds(t0, nt)], idx0)
    pltpu.sync_copy(pos1_hbm.at[pl.ds(t0, nt)], idx1)
    c0 = pltpu.async_copy(ys_hbm.at[idx0], buf0, sem0)
    c1 = pltpu.async_copy(ys_hbm.at[idx1], buf1, sem1)
    c0.wait()
    c1.wait()

    def row(r, _):
        for k in range(D // 16):
            s = pl.ds(k * 16, 16)
            buf0[r, s] = buf0[r, s] + buf1[r, s]
        return _
    lax.fori_loop(0, nt, row, 0)
    pltpu.sync_copy(buf0, out_hbm.at[pl.ds(t0, nt)])


def kernel(x, Wr, br, W1, b1, W2, b2):
    topi, topv = _router(x, Wr, br.reshape(1, E))
    ei = topi.reshape(-1)
    gv = topv.reshape(-1)
    idxs, gs, pos0, pos1, sinfo, _ = _dispatch(ei, gv)
    idxs3 = idxs[:PAD].reshape(NB, BT, 1)
    ys = _gmm(sinfo, idxs3, x.astype(jnp.bfloat16), W1,
              b1.reshape(E, 1, F), W2, b2.reshape(E, 1, D),
              gs.reshape(PAD, 1))
    return _combine(ys, pos0, pos1)


# speedup vs baseline: 1.3187x; 1.0041x over previous
import functools

import jax
import jax.numpy as jnp
from jax import lax
from jax.experimental import pallas as pl
from jax.experimental.pallas import tpu as pltpu
from jax.experimental.pallas import tpu_sc as plsc

T = 2048
E = 8
D = 768
F = 3072
KK = 2
A = T * KK
BT = 256
NB = A // BT + E - 1
PAD = NB * BT
PAD2 = PAD + 64
NTILE = 16
SEG = A // NTILE
RPW = PAD // 32

_MESH = plsc.VectorSubcoreMesh(core_axis_name="c", subcore_axis_name="s")


def _router_body(x_ref, wr_ref, br_ref, ti_ref, tv_ref):
    x = x_ref[...]
    logits = jnp.dot(x, wr_ref[...], preferred_element_type=jnp.float32)
    logits = logits + br_ref[...]
    lm = jnp.max(logits, axis=-1, keepdims=True)
    ex = jnp.exp(logits - lm)
    p = ex / jnp.sum(ex, axis=-1, keepdims=True)
    iota = lax.broadcasted_iota(jnp.int32, p.shape, 1)
    m1 = jnp.max(p, axis=-1, keepdims=True)
    i1 = jnp.min(jnp.where(p == m1, iota, E), axis=-1, keepdims=True)
    pneg = jnp.where(iota == i1, -jnp.inf, p)
    m2 = jnp.max(pneg, axis=-1, keepdims=True)
    i2 = jnp.min(jnp.where(pneg == m2, iota, E), axis=-1, keepdims=True)
    ti_ref[...] = jnp.concatenate([i1, i2], axis=1)
    tv_ref[...] = jnp.concatenate([m1, m2], axis=1)


def _router(x, Wr, br2):
    nt = T // BT
    return pl.pallas_call(
        _router_body,
        grid=(nt,),
        in_specs=[
            pl.BlockSpec((BT, D), lambda t: (t, 0)),
            pl.BlockSpec((D, E), lambda t: (0, 0)),
            pl.BlockSpec((1, E), lambda t: (0, 0)),
        ],
        out_specs=[
            pl.BlockSpec((BT, KK), lambda t: (t, 0)),
            pl.BlockSpec((BT, KK), lambda t: (t, 0)),
        ],
        out_shape=[
            jax.ShapeDtypeStruct((T, KK), jnp.int32),
            jax.ShapeDtypeStruct((T, KK), jnp.float32),
        ],
    )(x, Wr, br2)


@functools.partial(
    pl.kernel,
    out_type=(
        jax.ShapeDtypeStruct((PAD2,), jnp.int32),
        jax.ShapeDtypeStruct((PAD,), jnp.float32),
        jax.ShapeDtypeStruct((T,), jnp.int32),
        jax.ShapeDtypeStruct((T,), jnp.int32),
        jax.ShapeDtypeStruct((32,), jnp.int32),
        jax.ShapeDtypeStruct((NTILE, 16), jnp.int32),
    ),
    mesh=_MESH,
    compiler_params=pltpu.CompilerParams(needs_layout_passes=False),
    scratch_types=[
        pltpu.VMEM((SEG,), jnp.int32),
        pltpu.VMEM((2, SEG // 2), jnp.float32),
        pltpu.VMEM((16,), jnp.int32),
        pltpu.VMEM((NTILE, 16), jnp.int32),
        pltpu.VMEM((SEG,), jnp.int32),
        pltpu.VMEM((SEG,), jnp.int32),
        pltpu.VMEM((2, SEG // 2), jnp.int32),
        pltpu.VMEM((2, SEG // 2), jnp.int32),
        pltpu.VMEM((SEG // 2,), jnp.int32),
        pltpu.VMEM((SEG // 2,), jnp.int32),
        pltpu.VMEM((32,), jnp.int32),
    ],
)
def _dispatch(ei_hbm, gv_hbm, idxs_hbm, gs_hbm, pos0_hbm, pos1_hbm, sinfo_hbm,
              cnt_hbm, eiv, gvv, cnt_loc, cnts, tokl, posl, toks2, poss2,
              p0l, p1l, sinfo_loc):
    cid = lax.axis_index("c")
    sid = lax.axis_index("s")

    @pl.when(cid == 0)
    def _():
        iota = lax.iota(jnp.int32, 16)
        pltpu.sync_copy(ei_hbm.at[pl.ds(sid * SEG, SEG)], eiv)
        for j in range(2):
            pltpu.sync_copy(gv_hbm.at[pl.ds(sid * SEG + j * 128, 128)],
                            gvv.at[j])

        def _count_chunk(c, cntv):
            v = eiv[pl.ds(c * 16, 16)]
            for e in range(E):
                pc = plsc.all_reduce_population_count(v == e)
                cntv = cntv + jnp.where(iota == e, pc, 0)
            return cntv
        cnt_loc[...] = lax.fori_loop(0, SEG // 16, _count_chunk,
                                     jnp.zeros((16,), jnp.int32))
        pltpu.sync_copy(cnt_loc, cnt_hbm.at[sid])
        plsc.subcore_barrier()
        pltpu.sync_copy(cnt_hbm, cnts)

        tot = jnp.zeros((16,), jnp.int32)
        pre = jnp.zeros((16,), jnp.int32)
        for r in range(NTILE):
            row = cnts[r, :]
            tot = tot + row
            pre = pre + jnp.where(jnp.full((16,), r, jnp.int32) < sid, row, 0)
        n = [tot[e] for e in range(E)]
        pr = [pre[e] for e in range(E)]
        sp = [jnp.int32(0)]
        for e in range(E):
            nblk = (n[e] + (BT - 1)) // BT
            sp.append(sp[e] + nblk * BT)
        offs = [sp[e] + pr[e] for e in range(E)]
        nbu = sp[E] // BT

        def _pos_chunk(c, offs):
            offs = list(offs)
            v = eiv[pl.ds(c * 16, 16)]
            pos = jnp.zeros((16,), jnp.int32)
            for e in range(E):
                m = v == e
                pc = plsc.cumsum(jnp.where(m, 1, 0).astype(jnp.int32))
                pos = jnp.where(m, offs[e] + pc - 1, pos)
                cnt = plsc.all_reduce_population_count(m)
                offs[e] = offs[e] + cnt[0]
            pos = jnp.clip(pos, 0, PAD - 1)
            gi = sid * SEG + c * 16 + iota
            tokl[pl.ds(c * 16, 16)] = gi >> 1
            posl[pl.ds(c * 16, 16)] = pos
            lt = (c * 16 + iota) >> 1
            even = (gi & 1) == 0
            plsc.store_scatter(p0l, [lt], pos, mask=even)
            plsc.store_scatter(p1l, [lt], pos, mask=jnp.logical_not(even))
            return tuple(offs)
        lax.fori_loop(0, SEG // 16, _pos_chunk, tuple(offs))

        for j in range(2):
            for k in range(8):
                s = pl.ds(j * 128 + k * 16, 16)
                toks2[j, pl.ds(k * 16, 16)] = tokl[s]
                poss2[j, pl.ds(k * 16, 16)] = posl[s]
        for j in range(2):
            pltpu.sync_copy(toks2.at[j], idxs_hbm.at[poss2.at[j]])
            pltpu.sync_copy(gvv.at[j], gs_hbm.at[poss2.at[j]])
        pltpu.sync_copy(p0l, pos0_hbm.at[pl.ds(sid * (SEG // 2), SEG // 2)])
        pltpu.sync_copy(p1l, pos1_hbm.at[pl.ds(sid * (SEG // 2), SEG // 2)])

        @pl.when(sid == 0)
        def _():
            for c2 in range(2):
                bstart = (c2 * 16 + iota) * BT
                acc = jnp.zeros((16,), jnp.int32)
                for e in range(E):
                    acc = acc + (bstart >= sp[e + 1]).astype(jnp.int32)
                be = jnp.minimum(acc, E - 1)
                if c2 == 1:
                    be = jnp.where(iota == 15, nbu, be)
                sinfo_loc[pl.ds(c2 * 16, 16)] = be
            pltpu.sync_copy(sinfo_loc, sinfo_hbm)


def _gmm_body(sref, idx_ref, xh_ref, w1_ref, b1_ref, w2_ref, b2_ref, gs_ref,
              ys_ref, w1c, w2c):
    b = pl.program_id(0)

    @pl.when(b < sref[31])
    def _():
        prev = sref[jnp.maximum(b - 1, 0)]
        @pl.when((b == 0) | (sref[b] != prev))
        def _():
            w1c[...] = w1_ref[0].astype(jnp.bfloat16)
            w2c[...] = w2_ref[0].astype(jnp.bfloat16)

        idx = idx_ref[0]
        iota = lax.broadcasted_iota(jnp.int32, (BT, T), 1)
        oh = (idx == iota).astype(jnp.bfloat16)
        xsel = jnp.dot(oh, xh_ref[...],
                       preferred_element_type=jnp.float32)
        xsel = xsel.astype(jnp.bfloat16)
        h = jnp.dot(xsel, w1c[...], preferred_element_type=jnp.float32)
        h = jnp.maximum(h + b1_ref[0], 0.0).astype(jnp.bfloat16)
        y = jnp.dot(h, w2c[...], preferred_element_type=jnp.float32)
        ys_ref[...] = gs_ref[...] * (y + b2_ref[0])


def _gmm(sinfo, idxs3, xh, W1, b1r, W2, b2r, gs2):
    grid_spec = pltpu.PrefetchScalarGridSpec(
        num_scalar_prefetch=1,
        grid=(NB,),
        in_specs=[
            pl.BlockSpec((1, BT, 1), lambda b, s: (b, 0, 0)),
            pl.BlockSpec((T, D), lambda b, s: (0, 0)),
            pl.BlockSpec((1, D, F), lambda b, s: (s[b], 0, 0)),
            pl.BlockSpec((1, 1, F), lambda b, s: (s[b], 0, 0)),
            pl.BlockSpec((1, F, D), lambda b, s: (s[b], 0, 0)),
            pl.BlockSpec((1, 1, D), lambda b, s: (s[b], 0, 0)),
            pl.BlockSpec((BT, 1), lambda b, s: (b, 0)),
        ],
        out_specs=pl.BlockSpec((BT, D), lambda b, s: (b, 0)),
        scratch_shapes=[
            pltpu.VMEM((D, F), jnp.bfloat16),
            pltpu.VMEM((F, D), jnp.bfloat16),
        ],
    )
    return pl.pallas_call(
        _gmm_body,
        grid_spec=grid_spec,
        out_shape=jax.ShapeDtypeStruct((PAD, D), jnp.float32),
    )(sinfo, idxs3, xh, W1, b1r, W2, b2r, gs2)


@functools.partial(
    pl.kernel,
    out_type=jax.ShapeDtypeStruct((T, D), jnp.float32),
    mesh=_MESH,
    compiler_params=pltpu.CompilerParams(needs_layout_passes=False),
    scratch_types=[
        pltpu.VMEM((T // 32,), jnp.int32),
        pltpu.VMEM((T // 32,), jnp.int32),
        pltpu.VMEM((T // 32, D), jnp.float32),
        pltpu.VMEM((T // 32, D), jnp.float32),
        pltpu.SemaphoreType.DMA,
        pltpu.SemaphoreType.DMA,
    ],
)
def _combine(ys_hbm, pos0_hbm, pos1_hbm, out_hbm,
             idx0, idx1, buf0, buf1, sem0, sem1):
    wid = lax.axis_index("s") * 2 + lax.axis_index("c")
    nt = T // 32
    t0 = wid * nt
    pltpu.sync_copy(pos0_hbm.at[pl.ds(t0, nt)], idx0)
    pltpu.sync_copy(pos1_hbm.at[pl.ds(t0, nt)], idx1)
    c0 = pltpu.async_copy(ys_hbm.at[idx0], buf0, sem0)
    c1 = pltpu.async_copy(ys_hbm.at[idx1], buf1, sem1)
    c0.wait()
    c1.wait()

    def row(r, _):
        for k in range(D // 16):
            s = pl.ds(k * 16, 16)
            buf0[r, s] = buf0[r, s] + buf1[r, s]
        return _
    lax.fori_loop(0, nt, row, 0)
    pltpu.sync_copy(buf0, out_hbm.at[pl.ds(t0, nt)])


def kernel(x, Wr, br, W1, b1, W2, b2):
    topi, topv = _router(x, Wr, br.reshape(1, E))
    ei = topi.reshape(-1)
    gv = topv.reshape(-1)
    idxs, gs, pos0, pos1, sinfo, _ = _dispatch(ei, gv)
    idxs3 = idxs[:PAD].reshape(NB, BT, 1)
    ys = _gmm(sinfo, idxs3, x.astype(jnp.bfloat16), W1,
              b1.reshape(E, 1, F), W2, b2.reshape(E, 1, D),
              gs.reshape(PAD, 1))
    return _combine(ys, pos0, pos1)
